# Initial kernel scaffold; baseline (speedup 1.0000x reference)
import jax
import jax.numpy as jnp
from jax.experimental import pallas as pl


def kernel(x, edge_index, W1, b1, W2, b2, Wc, bc):
    raise NotImplementedError("write your pallas kernel here")


# speedup vs baseline: 6.8853x; 6.8853x over previous
import functools

import jax
import jax.numpy as jnp
from jax import lax
from jax.experimental import pallas as pl
from jax.experimental.pallas import tpu as pltpu
from jax.experimental.pallas import tpu_sc as plsc

N = 10000
FD = 128
EDGES = 320000
NCLS = 40

NC = 2
NS = 16
NW = NC * NS
EW = EDGES // NW
B = 80
NCHUNK = EW // B
NP = 10240
RT = NP // NS
ZR = 128
DW = 128

_MESH = plsc.VectorSubcoreMesh(core_axis_name="c", subcore_axis_name="s")


@functools.partial(
    pl.kernel,
    out_type=jax.ShapeDtypeStruct((2, NP, DW), jnp.float32),
    mesh=_MESH,
    scratch_types=[
        pltpu.VMEM((B,), jnp.int32),
        pltpu.VMEM((B, DW), jnp.float32),
        pltpu.VMEM((ZR, DW), jnp.float32),
        pltpu.VMEM_SHARED((NP, DW), jnp.float32),
    ],
)
def _sc_degree(dst_hbm, out_hbm, didx_v, ones_v, zb_v, deg_sh):
    c = lax.axis_index("c")
    s = lax.axis_index("s")
    wid = s * NC + c

    def _fill_ones(i, _):
        for j in range(DW // 16):
            ones_v[i, pl.ds(16 * j, 16)] = jnp.ones((16,), jnp.float32)
        return 0

    def _fill_zero(i, _):
        for j in range(DW // 16):
            zb_v[i, pl.ds(16 * j, 16)] = jnp.zeros((16,), jnp.float32)
        return 0

    lax.fori_loop(0, B, _fill_ones, 0)
    lax.fori_loop(0, ZR, _fill_zero, 0)

    r0 = s * RT
    for k in range(RT // ZR):
        pltpu.sync_copy(zb_v, deg_sh.at[pl.ds(r0 + k * ZR, ZR)])
    plsc.subcore_barrier()

    base = wid * EW

    def _chunk(i, _):
        off = pl.multiple_of(base + i * B, 8)
        pltpu.sync_copy(dst_hbm.at[pl.ds(off, B)], didx_v)
        pltpu.sync_copy(ones_v, deg_sh.at[didx_v], add=True)
        return 0

    lax.fori_loop(0, NCHUNK, _chunk, 0)
    plsc.subcore_barrier()
    pltpu.sync_copy(deg_sh.at[pl.ds(r0, RT)], out_hbm.at[c, pl.ds(r0, RT)])


@functools.partial(
    pl.kernel,
    out_type=jax.ShapeDtypeStruct((2, NP, FD), jnp.float32),
    mesh=_MESH,
    scratch_types=[
        pltpu.VMEM((B,), jnp.int32),
        pltpu.VMEM((B,), jnp.int32),
        pltpu.VMEM((B, FD), jnp.float32),
        pltpu.VMEM((ZR, FD), jnp.float32),
        pltpu.VMEM_SHARED((NP, FD), jnp.float32),
        pltpu.SemaphoreType.DMA,
    ],
)
def _sc_propagate(u_hbm, src_hbm, dst_hbm, out_hbm,
                  sidx_v, didx_v, rows_v, zb_v, acc_sh, sem):
    c = lax.axis_index("c")
    s = lax.axis_index("s")
    wid = s * NC + c

    def _zfill(i, _):
        for j in range(FD // 16):
            zb_v[i, pl.ds(16 * j, 16)] = jnp.zeros((16,), jnp.float32)
        return 0

    lax.fori_loop(0, ZR, _zfill, 0)

    r0 = s * RT
    for k in range(RT // ZR):
        pltpu.sync_copy(zb_v, acc_sh.at[pl.ds(r0 + k * ZR, ZR)])
    plsc.subcore_barrier()

    base = wid * EW

    def _chunk(i, _):
        off = pl.multiple_of(base + i * B, 8)
        pltpu.sync_copy(src_hbm.at[pl.ds(off, B)], sidx_v)
        pltpu.sync_copy(dst_hbm.at[pl.ds(off, B)], didx_v)
        pltpu.async_copy(u_hbm.at[sidx_v], rows_v, sem).wait()
        pltpu.sync_copy(rows_v, acc_sh.at[didx_v], add=True)
        return 0

    lax.fori_loop(0, NCHUNK, _chunk, 0)
    plsc.subcore_barrier()
    pltpu.sync_copy(acc_sh.at[pl.ds(r0, RT)], out_hbm.at[c, pl.ds(r0, RT)])


R = 2000
GRID = N // R


def _rows(width):
    return pl.BlockSpec((R, width), lambda i: (i, 0))


def _part(width, which):
    return pl.BlockSpec((1, R, width), lambda i, w=which: (w, i, 0))


def _full(shape):
    return pl.BlockSpec(shape, lambda i: (0,) * len(shape))


def _prep_body(x_ref, dega_ref, degb_ref, w_ref, y_ref, u_ref, d_ref):
    deg = dega_ref[0, :, 0:1] + degb_ref[0, :, 0:1]
    dinv = jnp.where(deg > 0.0, lax.rsqrt(jnp.maximum(deg, 1e-12)), 0.0)
    dinvb = jnp.broadcast_to(dinv, (R, FD))
    x = x_ref[...]
    y_ref[...] = jnp.dot(x, w_ref[...], preferred_element_type=jnp.float32)
    u_ref[...] = dinvb * x
    d_ref[...] = dinvb


_tc_prep = pl.pallas_call(
    _prep_body,
    grid=(GRID,),
    in_specs=[_rows(FD), _part(DW, 0), _part(DW, 1), _full((FD, FD))],
    out_specs=[_rows(FD), _rows(FD), _rows(FD)],
    out_shape=[jax.ShapeDtypeStruct((N, FD), jnp.float32)] * 3,
)


def _step_body(pa_ref, pb_ref, d_ref, w_ref, yin_ref, y_ref, u_ref):
    d = d_ref[...]
    h = d * (pa_ref[0] + pb_ref[0])
    y_ref[...] = yin_ref[...] + jnp.dot(
        h, w_ref[...], preferred_element_type=jnp.float32)
    u_ref[...] = d * h


_tc_step = pl.pallas_call(
    _step_body,
    grid=(GRID,),
    in_specs=[_part(FD, 0), _part(FD, 1), _rows(FD), _full((FD, FD)), _rows(FD)],
    out_specs=[_rows(FD), _rows(FD)],
    out_shape=[jax.ShapeDtypeStruct((N, FD), jnp.float32)] * 2,
)


def _bridge_body(pa_ref, pb_ref, d_ref, w_ref, yin_ref, b_ref, wn_ref,
                 y_ref, u_ref):
    d = d_ref[...]
    h = d * (pa_ref[0] + pb_ref[0])
    a = jnp.maximum(
        yin_ref[...]
        + jnp.dot(h, w_ref[...], preferred_element_type=jnp.float32)
        + b_ref[...], 0.0)
    y_ref[...] = jnp.dot(a, wn_ref[...], preferred_element_type=jnp.float32)
    u_ref[...] = d * a


_tc_bridge = pl.pallas_call(
    _bridge_body,
    grid=(GRID,),
    in_specs=[_part(FD, 0), _part(FD, 1), _rows(FD), _full((FD, FD)), _rows(FD),
              _full((1, FD)), _full((FD, FD))],
    out_specs=[_rows(FD), _rows(FD)],
    out_shape=[jax.ShapeDtypeStruct((N, FD), jnp.float32)] * 2,
)


def _final_body(pa_ref, pb_ref, d_ref, w_ref, yin_ref, b_ref, wc_ref, bc_ref,
                o_ref):
    d = d_ref[...]
    h = d * (pa_ref[0] + pb_ref[0])
    a = jnp.maximum(
        yin_ref[...]
        + jnp.dot(h, w_ref[...], preferred_element_type=jnp.float32)
        + b_ref[...], 0.0)
    o_ref[...] = jnp.dot(
        a, wc_ref[...], preferred_element_type=jnp.float32) + bc_ref[...]


_tc_final = pl.pallas_call(
    _final_body,
    grid=(GRID,),
    in_specs=[_part(FD, 0), _part(FD, 1), _rows(FD), _full((FD, FD)), _rows(FD),
              _full((1, FD)), _full((FD, NCLS)), _full((1, NCLS))],
    out_specs=_rows(NCLS),
    out_shape=jax.ShapeDtypeStruct((N, NCLS), jnp.float32),
)


def kernel(x, edge_index, W1, b1, W2, b2, Wc, bc):
    ei = edge_index.astype(jnp.int32)
    src = ei[0]
    dst = ei[1]

    degp = _sc_degree(dst)
    y, u, dinvb = _tc_prep(x, degp, degp, W1[0])

    for k in (1, 2):
        p = _sc_propagate(u, src, dst)
        y, u = _tc_step(p, p, dinvb, W1[k], y)
    p = _sc_propagate(u, src, dst)
    y, u = _tc_bridge(p, p, dinvb, W1[3], y, b1.reshape(1, FD), W2[0])

    for k in (1, 2):
        p = _sc_propagate(u, src, dst)
        y, u = _tc_step(p, p, dinvb, W2[k], y)
    p = _sc_propagate(u, src, dst)
    return _tc_final(p, p, dinvb, W2[3], y, b2.reshape(1, FD),
                     Wc, bc.reshape(1, NCLS))
